# baseline (speedup 1.0000x reference)
import functools

import jax
import jax.numpy as jnp
from jax import lax
from jax.experimental import pallas as pl
from jax.experimental.pallas import tpu as pltpu
from jax.experimental.pallas import tpu_sc as plsc

_D = 14336
_V = 128
_SPLIT = 80
_ROWS = 8192
_NC = 2
_NS = 16
_NW = _NC * _NS
_NA = 20
_NB = _NW - _NA
_CAP = 704
_NSA = 16
_NBB = 4


def _direct_machine(table_sp, out_hbm, pos_v, val_v, sems, groups):
    def group(g, first):
        pvec = pos_v[pl.ds(16 * g, 16)]
        vvec = val_v[pl.ds(16 * g, 16)]
        for k in range(16):
            if not first:
                pltpu.make_async_copy(
                    table_sp.at[pl.ds(0, 1)],
                    out_hbm.at[pl.ds(0, 1)], sems[k]).wait()
            pltpu.make_async_copy(
                table_sp.at[pl.ds(vvec[k], 1)],
                out_hbm.at[pl.ds(pvec[k], 1)], sems[k]).start()

    @pl.when(groups > 0)
    def _():
        group(0, True)

    def body(j, carry):
        @pl.when(j + 1 < groups)
        def _():
            group(j + 1, False)
        return carry

    lax.fori_loop(0, _CAP // 16 - 1, body, 0)

    @pl.when(groups > 0)
    def _():
        for k in range(16):
            pltpu.make_async_copy(
                table_sp.at[pl.ds(0, 1)],
                out_hbm.at[pl.ds(0, 1)], sems[k]).wait()


def _staged_machine(table_hbm, out_hbm, pos_v, val_v, bufs, gsems, ssems,
                    iters):
    def iteration(t, first):
        pvec = pos_v[pl.ds(8 * t, 16)]
        vvec = val_v[pl.ds(8 * t, 16)]
        for r in range(2):
            for b in range(_NBB):
                if not (first and r == 0):
                    pltpu.make_async_copy(
                        bufs[b], out_hbm.at[pl.ds(0, 1)], ssems[b]).wait()
                pltpu.make_async_copy(
                    table_hbm.at[pl.ds(vvec[4 * r + b], 1)],
                    bufs[b], gsems[b]).start()
            for b in range(_NBB):
                pltpu.make_async_copy(
                    table_hbm.at[pl.ds(0, 1)], bufs[b], gsems[b]).wait()
                pltpu.make_async_copy(
                    bufs[b], out_hbm.at[pl.ds(pvec[4 * r + b], 1)],
                    ssems[b]).start()

    @pl.when(iters > 0)
    def _():
        iteration(0, True)

    def body(j, carry):
        @pl.when(j + 1 < iters)
        def _():
            iteration(j + 1, False)
        return carry

    lax.fori_loop(0, _CAP // 8 - 1, body, 0)

    @pl.when(iters > 0)
    def _():
        for b in range(_NBB):
            pltpu.make_async_copy(
                bufs[b], out_hbm.at[pl.ds(0, 1)], ssems[b]).wait()


def _sc_body(table_hbm, pos_hbm, val_hbm, cnt_hbm, out_hbm,
             pos_v, val_v, cnt_v, table_sp, b0, b1, b2, b3, *sems):
    sid = lax.axis_index("s")
    wid = sid * _NC + lax.axis_index("c")

    asems = sems[:_NSA]
    gsems = sems[_NSA:_NSA + _NBB]
    ssems = sems[_NSA + _NBB:_NSA + 2 * _NBB]

    @pl.when(sid < _SPLIT // 8)
    def _():
        pltpu.sync_copy(table_hbm.at[pl.ds(sid * 8, 8)],
                        table_sp.at[pl.ds(sid * 8, 8)])

    pltpu.sync_copy(pos_hbm.at[wid], pos_v)
    pltpu.sync_copy(val_hbm.at[wid], val_v)
    pltpu.sync_copy(cnt_hbm.at[wid], cnt_v)
    plsc.subcore_barrier()

    n = cnt_v[pl.ds(0, 16)][0]

    @pl.when(wid < _NA)
    def _():
        _direct_machine(table_sp, out_hbm, pos_v, val_v, asems, n)

    @pl.when(wid >= _NA)
    def _():
        _staged_machine(table_hbm, out_hbm, pos_v, val_v,
                        (b0, b1, b2, b3), gsems, ssems, n)


@functools.partial(
    pl.kernel,
    mesh=plsc.VectorSubcoreMesh(core_axis_name="c", subcore_axis_name="s"),
    out_type=jax.ShapeDtypeStruct((_ROWS, _D), jnp.float32),
    scratch_types=(
        [pltpu.VMEM((_CAP,), jnp.int32)] * 2
        + [pltpu.VMEM((16,), jnp.int32),
           pltpu.VMEM_SHARED((_SPLIT, _D), jnp.float32)]
        + [pltpu.VMEM((1, _D), jnp.float32)] * _NBB
        + [pltpu.SemaphoreType.DMA] * (_NSA + 2 * _NBB)
    ),
)
def _sc_gather(table_hbm, pos_hbm, val_hbm, cnt_hbm, out_hbm, *rest):
    _sc_body(table_hbm, pos_hbm, val_hbm, cnt_hbm, out_hbm, *rest)


def _ceil_to(x, m):
    return (x + m - 1) // m * m


@jax.jit
def kernel(prefix, table):
    b, s = prefix.shape
    idx = prefix.reshape(_ROWS).astype(jnp.int32)

    is_b = (idx >= _SPLIT).astype(jnp.int32)
    order = jnp.argsort(is_b, stable=True).astype(jnp.int32)
    sv = jnp.take(idx, order)
    sp = order
    cnt_a = jnp.sum(1 - is_b)
    cnt_b = _ROWS - cnt_a
    qa = _ceil_to((cnt_a + _NA - 1) // _NA, 16)
    qb = _ceil_to((cnt_b + _NB - 1) // _NB, 8)

    t = jnp.arange(_NW, dtype=jnp.int32)[:, None]
    i = jnp.arange(_CAP, dtype=jnp.int32)[None, :]
    a_g = jnp.clip(t * qa + i, 0, jnp.maximum(cnt_a - 1, 0))
    b_g = jnp.clip(cnt_a + (t - _NA) * qb + i,
                   jnp.minimum(cnt_a, _ROWS - 1), _ROWS - 1)
    g = jnp.where(t < _NA, a_g, b_g)
    pos = jnp.take(sp, g)
    val = jnp.take(sv, g)
    na_grp = jnp.where(cnt_a > 0, qa // 16, 0)
    nb_it = jnp.where(cnt_b > 0, qb // 8, 0)
    per_tile = jnp.where(t[:, 0] < _NA, na_grp, nb_it)
    cnt = jnp.tile(per_tile[:, None], (1, 16)).astype(jnp.int32)

    out = _sc_gather(table, pos, val, cnt)
    return out.reshape(b, s, _D)


# speedup vs baseline: 1.5311x; 1.5311x over previous
import functools

import jax
import jax.numpy as jnp
from jax import lax
from jax.experimental import pallas as pl
from jax.experimental.pallas import tpu as pltpu
from jax.experimental.pallas import tpu_sc as plsc

_D = 14336
_V = 128
_SPLIT = 104
_ROWS = 8192
_NC = 2
_NS = 16
_NW = _NC * _NS
_NA = 16
_NB = _NW - _NA
_CAP = 528
_NSA = 16
_NBB = 2


def _direct_machine(table_sp, out_hbm, pos_v, val_v, sems, groups):
    def group(g, first):
        pvec = pos_v[pl.ds(16 * g, 16)]
        vvec = val_v[pl.ds(16 * g, 16)]
        for k in range(16):
            if not first:
                pltpu.make_async_copy(
                    table_sp.at[pl.ds(0, 1)],
                    out_hbm.at[pl.ds(0, 1)], sems[k]).wait()
            pltpu.make_async_copy(
                table_sp.at[pl.ds(vvec[k], 1)],
                out_hbm.at[pl.ds(pvec[k], 1)], sems[k]).start()

    @pl.when(groups > 0)
    def _():
        group(0, True)

    def body(j, carry):
        @pl.when(j + 1 < groups)
        def _():
            group(j + 1, False)
        return carry

    lax.fori_loop(0, _CAP // 16 - 1, body, 0)

    @pl.when(groups > 0)
    def _():
        for k in range(16):
            pltpu.make_async_copy(
                table_sp.at[pl.ds(0, 1)],
                out_hbm.at[pl.ds(0, 1)], sems[k]).wait()


def _staged_machine(table_hbm, out_hbm, pos_v, val_v, bufs, gsems, ssems,
                    iters):
    def wait_gather(b):
        pltpu.make_async_copy(
            table_hbm.at[pl.ds(0, 1)], bufs[b], gsems[b]).wait()

    def wait_scatter(b):
        pltpu.make_async_copy(
            bufs[b], out_hbm.at[pl.ds(0, 1)], ssems[b]).wait()

    def iteration(t, first):
        pvec = pos_v[pl.ds(8 * t, 16)]
        vvec = val_v[pl.ds(8 * t, 16)]
        if not first:
            ppvec = pos_v[pl.ds(8 * t - 8, 16)]
            wait_gather(1)
            pltpu.make_async_copy(
                bufs[1], out_hbm.at[pl.ds(ppvec[7], 1)], ssems[1]).start()
        for k in range(8):
            b = k % 2
            if not (first and k < 2):
                wait_scatter(b)
            pltpu.make_async_copy(
                table_hbm.at[pl.ds(vvec[k], 1)], bufs[b], gsems[b]).start()
            if k >= 1:
                b1 = (k - 1) % 2
                wait_gather(b1)
                pltpu.make_async_copy(
                    bufs[b1], out_hbm.at[pl.ds(pvec[k - 1], 1)],
                    ssems[b1]).start()

    @pl.when(iters > 0)
    def _():
        iteration(0, True)

    def body(j, carry):
        @pl.when(j + 1 < iters)
        def _():
            iteration(j + 1, False)
        return carry

    lax.fori_loop(0, _CAP // 8 - 1, body, 0)

    @pl.when(iters > 0)
    def _():
        lpvec = pos_v[pl.ds(8 * iters - 8, 16)]
        wait_gather(1)
        pltpu.make_async_copy(
            bufs[1], out_hbm.at[pl.ds(lpvec[7], 1)], ssems[1]).start()
        for b in range(_NBB):
            wait_scatter(b)


def _sc_body(table_hbm, pos_hbm, val_hbm, cnt_hbm, out_hbm,
             pos_v, val_v, cnt_v, table_sp, b0, b1, *sems):
    sid = lax.axis_index("s")
    wid = sid * _NC + lax.axis_index("c")

    asems = sems[:_NSA]
    gsems = sems[_NSA:_NSA + _NBB]
    ssems = sems[_NSA + _NBB:_NSA + 2 * _NBB]

    @pl.when(sid < _SPLIT // 8)
    def _():
        pltpu.sync_copy(table_hbm.at[pl.ds(sid * 8, 8)],
                        table_sp.at[pl.ds(sid * 8, 8)])

    pltpu.sync_copy(pos_hbm.at[wid], pos_v)
    pltpu.sync_copy(val_hbm.at[wid], val_v)
    pltpu.sync_copy(cnt_hbm.at[wid], cnt_v)
    plsc.subcore_barrier()

    n = cnt_v[pl.ds(0, 16)][0]

    @pl.when(wid < _NA)
    def _():
        _direct_machine(table_sp, out_hbm, pos_v, val_v, asems, n)

    @pl.when(wid >= _NA)
    def _():
        _staged_machine(table_hbm, out_hbm, pos_v, val_v,
                        (b0, b1), gsems, ssems, n)


@functools.partial(
    pl.kernel,
    mesh=plsc.VectorSubcoreMesh(core_axis_name="c", subcore_axis_name="s"),
    out_type=jax.ShapeDtypeStruct((_ROWS, _D), jnp.float32),
    scratch_types=(
        [pltpu.VMEM((_CAP,), jnp.int32)] * 2
        + [pltpu.VMEM((16,), jnp.int32),
           pltpu.VMEM_SHARED((_SPLIT, _D), jnp.float32)]
        + [pltpu.VMEM((1, _D), jnp.float32)] * _NBB
        + [pltpu.SemaphoreType.DMA] * (_NSA + 2 * _NBB)
    ),
)
def _sc_gather(table_hbm, pos_hbm, val_hbm, cnt_hbm, out_hbm, *rest):
    _sc_body(table_hbm, pos_hbm, val_hbm, cnt_hbm, out_hbm, *rest)


def _ceil_to(x, m):
    return (x + m - 1) // m * m


@jax.jit
def kernel(prefix, table):
    b, s = prefix.shape
    idx = prefix.reshape(_ROWS).astype(jnp.int32)

    is_b = (idx >= _SPLIT).astype(jnp.int32)
    order = jnp.argsort(is_b, stable=True).astype(jnp.int32)
    sv = jnp.take(idx, order)
    sp = order
    cnt_a = jnp.sum(1 - is_b)
    cnt_b = _ROWS - cnt_a
    qa = _ceil_to((cnt_a + _NA - 1) // _NA, 16)
    qb = _ceil_to((cnt_b + _NB - 1) // _NB, 8)

    t = jnp.arange(_NW, dtype=jnp.int32)[:, None]
    i = jnp.arange(_CAP, dtype=jnp.int32)[None, :]
    a_g = jnp.clip(t * qa + i, 0, jnp.maximum(cnt_a - 1, 0))
    b_g = jnp.clip(cnt_a + (t - _NA) * qb + i,
                   jnp.minimum(cnt_a, _ROWS - 1), _ROWS - 1)
    g = jnp.where(t < _NA, a_g, b_g)
    pos = jnp.take(sp, g)
    val = jnp.take(sv, g)
    na_grp = jnp.where(cnt_a > 0, qa // 16, 0)
    nb_it = jnp.where(cnt_b > 0, qb // 8, 0)
    per_tile = jnp.where(t[:, 0] < _NA, na_grp, nb_it)
    cnt = jnp.tile(per_tile[:, None], (1, 16)).astype(jnp.int32)

    out = _sc_gather(table, pos, val, cnt)
    return out.reshape(b, s, _D)


# speedup vs baseline: 2.6045x; 1.7011x over previous
import functools

import jax
import jax.numpy as jnp
from jax import lax
from jax.experimental import pallas as pl
from jax.experimental.pallas import tpu as pltpu
from jax.experimental.pallas import tpu_sc as plsc

_D = 14336
_V = 128
_ROWS = 8192
_NC = 2
_NS = 16
_NW = _NC * _NS
_RPW = _ROWS // _NW
_NSEM = 16


def _sc_body(table_hbm, idx_hbm, out_hbm, idx_v, table_sp, *sems):
    sid = lax.axis_index("s")
    wid = sid * _NC + lax.axis_index("c")
    base = wid * _RPW

    rows_per_tile = _V // _NS
    pltpu.sync_copy(table_hbm.at[pl.ds(sid * rows_per_tile, rows_per_tile)],
                    table_sp.at[pl.ds(sid * rows_per_tile, rows_per_tile)])
    pltpu.sync_copy(idx_hbm.at[pl.ds(base, _RPW)], idx_v)
    plsc.subcore_barrier()

    def start(v, r, k):
        pltpu.make_async_copy(
            table_sp.at[pl.ds(v, 1)],
            out_hbm.at[pl.ds(base + r, 1)], sems[k]).start()

    def wait(k):
        pltpu.make_async_copy(
            table_sp.at[pl.ds(0, 1)],
            out_hbm.at[pl.ds(base, 1)], sems[k]).wait()

    vec0 = idx_v[pl.ds(0, _NSEM)]
    for k in range(_NSEM):
        start(vec0[k], k, k)

    def body(j, carry):
        vec = idx_v[pl.ds(_NSEM * (j + 1), _NSEM)]
        for k in range(_NSEM):
            wait(k)
            start(vec[k], _NSEM * (j + 1) + k, k)
        return carry

    lax.fori_loop(0, _RPW // _NSEM - 1, body, 0)

    for k in range(_NSEM):
        wait(k)


@functools.partial(
    pl.kernel,
    mesh=plsc.VectorSubcoreMesh(core_axis_name="c", subcore_axis_name="s"),
    out_type=jax.ShapeDtypeStruct((_ROWS, _D), jnp.float32),
    scratch_types=(
        [pltpu.VMEM((_RPW,), jnp.int32),
         pltpu.VMEM_SHARED((_V, _D), jnp.float32)]
        + [pltpu.SemaphoreType.DMA] * _NSEM
    ),
)
def _sc_gather(table_hbm, idx_hbm, out_hbm, *scratch):
    _sc_body(table_hbm, idx_hbm, out_hbm, *scratch)


@jax.jit
def kernel(prefix, table):
    b, s = prefix.shape
    idx = prefix.reshape(_ROWS).astype(jnp.int32)
    out = _sc_gather(table, idx)
    return out.reshape(b, s, _D)
